# Initial kernel scaffold; baseline (speedup 1.0000x reference)
import jax
import jax.numpy as jnp
from jax.experimental import pallas as pl


def kernel(x, edge_index, batch, W1, b1, W2, b2, W3, b3, lin1_W, lin1_b, lin2_W, lin2_b):
    raise NotImplementedError("write your pallas kernel here")


# speedup vs baseline: 16.7864x; 16.7864x over previous
import functools

import jax
import jax.numpy as jnp
from jax import lax
from jax.experimental import pallas as pl
from jax.experimental.pallas import tpu as pltpu
from jax.experimental.pallas import tpu_sc as plsc

N = 10000
E = 320000
D = 128
H = 128
C = 10
G = 64

NC = 2
NS = 16
NW = NC * NS

EW = E // NW
SEG = 80
WSEG = EW // SEG
R0 = 624
R1 = N - (NS - 1) * R0

RB = 2000

_MESH = plsc.VectorSubcoreMesh(
    core_axis_name="c", subcore_axis_name="s", num_cores=NC, num_subcores=NS)


DW = 8


@functools.partial(
    pl.kernel,
    out_type=jax.ShapeDtypeStruct((NC, N, DW), jnp.float32),
    mesh=_MESH,
    scratch_types=[
        pltpu.VMEM((WSEG, SEG), jnp.int32),
        pltpu.VMEM((SEG, DW), jnp.float32),
        pltpu.VMEM_SHARED((N, DW), jnp.float32),
    ],
)
def _deg_sc(dst_hbm, zeros_hbm, ones_hbm, out_hbm, dst_v, ones_v, acc):
  c = lax.axis_index("c")
  s = lax.axis_index("s")
  wid = c * NS + s
  pltpu.sync_copy(dst_hbm.at[wid], dst_v)
  pltpu.sync_copy(ones_hbm, ones_v)

  @pl.when(s == 0)
  def _():
    pltpu.sync_copy(zeros_hbm, acc)

  plsc.subcore_barrier()

  def body(j, carry):
    pltpu.sync_copy(ones_v, acc.at[dst_v.at[j]], add=True)
    return carry

  lax.fori_loop(0, WSEG, body, 0)
  plsc.subcore_barrier()

  @pl.when(s == 0)
  def _():
    pltpu.sync_copy(acc, out_hbm.at[c])


@functools.partial(
    pl.kernel,
    out_type=jax.ShapeDtypeStruct((NC, N, H), jnp.float32),
    mesh=_MESH,
    scratch_types=[
        pltpu.VMEM((WSEG, SEG), jnp.int32),
        pltpu.VMEM((WSEG, SEG), jnp.int32),
        pltpu.VMEM((SEG, H), jnp.float32),
        pltpu.VMEM_SHARED((N, H), jnp.float32),
        pltpu.SemaphoreType.DMA,
    ],
)
def _agg_sc(hs_hbm, src_hbm, dst_hbm, zeros_hbm, out_hbm,
            src_v, dst_v, rows_v, acc, sem):
  c = lax.axis_index("c")
  s = lax.axis_index("s")
  wid = c * NS + s
  pltpu.sync_copy(src_hbm.at[wid], src_v)
  pltpu.sync_copy(dst_hbm.at[wid], dst_v)

  @pl.when(s < NS - 1)
  def _():
    pltpu.sync_copy(zeros_hbm.at[pl.ds(s * R0, R0)], acc.at[pl.ds(s * R0, R0)])

  @pl.when(s == NS - 1)
  def _():
    pltpu.sync_copy(zeros_hbm.at[pl.ds((NS - 1) * R0, R1)],
                    acc.at[pl.ds((NS - 1) * R0, R1)])

  plsc.subcore_barrier()

  def body(j, carry):
    pltpu.async_copy(hs_hbm.at[src_v.at[j]], rows_v, sem).wait()
    pltpu.sync_copy(rows_v, acc.at[dst_v.at[j]], add=True)
    return carry

  lax.fori_loop(0, WSEG, body, 0)
  plsc.subcore_barrier()

  @pl.when(s < NS - 1)
  def _():
    pltpu.sync_copy(acc.at[pl.ds(s * R0, R0)],
                    out_hbm.at[c, pl.ds(s * R0, R0)])

  @pl.when(s == NS - 1)
  def _():
    pltpu.sync_copy(acc.at[pl.ds((NS - 1) * R0, R1)],
                    out_hbm.at[c, pl.ds((NS - 1) * R0, R1)])


def _mm1_body(x_ref, w_ref, degp_ref, hs_ref, dinv_ref):
  p = degp_ref[...]
  dinv = lax.rsqrt(1.0 + p[0, :, 0:1] + p[1, :, 0:1])
  h = jnp.dot(x_ref[...], w_ref[...], preferred_element_type=jnp.float32)
  hs_ref[...] = h * dinv
  dinv_ref[...] = dinv


def _mm1(x, W1, degp):
  return pl.pallas_call(
      _mm1_body,
      grid=(N // RB,),
      in_specs=[
          pl.BlockSpec((RB, D), lambda i: (i, 0)),
          pl.BlockSpec((D, H), lambda i: (0, 0)),
          pl.BlockSpec((NC, RB, DW), lambda i: (0, i, 0)),
      ],
      out_specs=[
          pl.BlockSpec((RB, H), lambda i: (i, 0)),
          pl.BlockSpec((RB, 1), lambda i: (i, 0)),
      ],
      out_shape=[
          jax.ShapeDtypeStruct((N, H), jnp.float32),
          jax.ShapeDtypeStruct((N, 1), jnp.float32),
      ],
  )(x, W1, degp)


def _fuse_body(acc_ref, hs_ref, dinv_ref, b_ref, w_ref, out_ref):
  a = acc_ref[...]
  dinv = dinv_ref[...]
  t = (a[0] + a[1] + hs_ref[...]) * dinv + b_ref[...]
  t = jnp.maximum(t, 0.0)
  out_ref[...] = jnp.dot(t, w_ref[...],
                         preferred_element_type=jnp.float32) * dinv


def _fuse(acc, hs, dinv, b, W):
  return pl.pallas_call(
      _fuse_body,
      grid=(N // RB,),
      in_specs=[
          pl.BlockSpec((NC, RB, H), lambda i: (0, i, 0)),
          pl.BlockSpec((RB, H), lambda i: (i, 0)),
          pl.BlockSpec((RB, 1), lambda i: (i, 0)),
          pl.BlockSpec((1, H), lambda i: (0, 0)),
          pl.BlockSpec((H, H), lambda i: (0, 0)),
      ],
      out_specs=pl.BlockSpec((RB, H), lambda i: (i, 0)),
      out_shape=jax.ShapeDtypeStruct((N, H), jnp.float32),
  )(acc, hs, dinv, b, W)


def _head_body(acc_ref, hs_ref, dinv_ref, b_ref, batch_ref,
               w1_ref, c1_ref, w2_ref, c2_ref, out_ref, sums, cnt):
  i = pl.program_id(0)

  @pl.when(i == 0)
  def _():
    sums[...] = jnp.zeros_like(sums)
    cnt[...] = jnp.zeros_like(cnt)

  a = acc_ref[...]
  dinv = dinv_ref[...]
  u = (a[0] + a[1] + hs_ref[...]) * dinv + b_ref[...]
  u = jnp.maximum(u, 0.0)
  gids = lax.broadcasted_iota(jnp.int32, (1, G), 1)
  mask = (batch_ref[...] == gids).astype(jnp.float32)
  dn = (((0,), (0,)), ((), ()))
  sums[...] += lax.dot_general(mask, u, dn, preferred_element_type=jnp.float32)
  cnt[...] += lax.dot_general(mask, jnp.ones_like(u), dn,
                              preferred_element_type=jnp.float32)

  @pl.when(i == pl.num_programs(0) - 1)
  def _():
    pooled = sums[...] / jnp.maximum(cnt[...], 1.0)
    t = jnp.dot(pooled, w1_ref[...], preferred_element_type=jnp.float32)
    t = jnp.maximum(t + c1_ref[...], 0.0)
    out_ref[...] = jnp.dot(t, w2_ref[...],
                           preferred_element_type=jnp.float32) + c2_ref[...]


def _head(acc, hs, dinv, b, batch2d, lin1_W, lin1_b, lin2_W, lin2_b):
  return pl.pallas_call(
      _head_body,
      grid=(N // RB,),
      in_specs=[
          pl.BlockSpec((NC, RB, H), lambda i: (0, i, 0)),
          pl.BlockSpec((RB, H), lambda i: (i, 0)),
          pl.BlockSpec((RB, 1), lambda i: (i, 0)),
          pl.BlockSpec((1, H), lambda i: (0, 0)),
          pl.BlockSpec((RB, 1), lambda i: (i, 0)),
          pl.BlockSpec((H, H), lambda i: (0, 0)),
          pl.BlockSpec((1, H), lambda i: (0, 0)),
          pl.BlockSpec((H, C), lambda i: (0, 0)),
          pl.BlockSpec((1, C), lambda i: (0, 0)),
      ],
      out_specs=pl.BlockSpec((G, C), lambda i: (0, 0)),
      out_shape=jax.ShapeDtypeStruct((G, C), jnp.float32),
      scratch_shapes=[
          pltpu.VMEM((G, H), jnp.float32),
          pltpu.VMEM((G, H), jnp.float32),
      ],
  )(acc, hs, dinv, b, batch2d, lin1_W, lin1_b, lin2_W, lin2_b)


def kernel(x, edge_index, batch, W1, b1, W2, b2, W3, b3,
           lin1_W, lin1_b, lin2_W, lin2_b):
  src = edge_index[0].reshape(NW, WSEG, SEG)
  dst = edge_index[1].reshape(NW, WSEG, SEG)
  zeros2d = jnp.zeros((N, H), jnp.float32)
  zeros8 = jnp.zeros((N, DW), jnp.float32)
  ones8 = jnp.ones((SEG, DW), jnp.float32)

  degp = _deg_sc(dst, zeros8, ones8)
  hs, dinv = _mm1(x, W1, degp)

  acc = _agg_sc(hs, src, dst, zeros2d)
  hs = _fuse(acc, hs, dinv, b1.reshape(1, H), W2)
  acc = _agg_sc(hs, src, dst, zeros2d)
  hs = _fuse(acc, hs, dinv, b2.reshape(1, H), W3)
  acc = _agg_sc(hs, src, dst, zeros2d)

  return _head(acc, hs, dinv, b3.reshape(1, H), batch.reshape(N, 1),
               lin1_W, lin1_b.reshape(1, H), lin2_W, lin2_b.reshape(1, C))
